# baseline (speedup 1.0000x reference)
import jax
import jax.numpy as jnp
import numpy as np
from jax import lax
from jax.experimental import pallas as pl
from jax.experimental.pallas import tpu as pltpu
from jax.experimental.pallas import tpu_sc as plsc

CHEB_ORDER = 30
T_SCALE = 5.0
N_NODES = 10000
N_EDGES = 320000
D_FEAT = 128
LAMBDA_MAX = 2.0

NC = 2
NS = 16
NW = NC * NS
CHUNK = 64
NCH = 5120
CPT = NCH // NW
EPAD = NCH * CHUNK
IB = 16
NBLK = CPT // IB
NBUF = 4
SPL = 2
ACC_R = 10112
RCH = 40
NRCH = N_NODES // RCH
ITER = (NRCH + NW - 1) // NW


def _cheb(M, t_scale, lambda_max=LAMBDA_MAX):
    j = np.arange(M, dtype=np.float64)
    x = np.cos(np.pi * (j + 0.5) / M)
    lambdas = lambda_max / 2.0 * (x + 1.0)
    f_vals = np.exp(-t_scale * lambdas)
    coeffs = np.zeros(M, dtype=np.float64)
    for k in range(M):
        T_k_x = np.cos(k * np.arccos(x))
        coeffs[k] = 2.0 / M * np.sum(f_vals * T_k_x)
    coeffs[0] /= 2.0
    return coeffs.astype(np.float32)


def _spmv_body(col_hbm, row_hbm, val_hbm, v_hbm, p_hbm,
               col_v, row_v, val_v, ring, sem, acc_sh):
    cid = lax.axis_index("c")
    sid = lax.axis_index("s")
    tile = cid * NS + sid

    zbuf = ring.at[pl.ds(0, RCH)]

    def _zr(r, _):
        for d in range(D_FEAT // 16):
            zbuf[r, pl.ds(d * 16, 16)] = jnp.zeros((16,), jnp.float32)
        return _

    lax.fori_loop(0, RCH, _zr, None)

    def _zc(i, _):
        j = sid + NS * i

        @pl.when(j * RCH < ACC_R)
        def _():
            off = pl.multiple_of(j * RCH, 8)
            pltpu.sync_copy(zbuf, acc_sh.at[pl.ds(off, RCH)])
        return _

    lax.fori_loop(0, (ACC_R // RCH + NS - 1) // NS, _zc, None)

    plsc.subcore_barrier()

    bufs = [ring.at[pl.ds(b * CHUNK, CHUNK)] for b in range(NBUF)]
    sems = [sem.at[b] for b in range(NBUF)]
    tbase = pl.multiple_of(tile * CPT, 8)

    def _blk(bi, _):
        boff = pl.multiple_of(tbase + bi * IB, 8)
        pltpu.sync_copy(col_hbm.at[pl.ds(boff, IB)], col_v)
        pltpu.sync_copy(row_hbm.at[pl.ds(boff, IB)], row_v)
        pltpu.sync_copy(val_hbm.at[pl.ds(boff, IB)], val_v)

        def _gstart(j, bj):
            for t in range(SPL):
                pltpu.async_copy(
                    v_hbm.at[col_v.at[j, pl.ds(t * (CHUNK // SPL), CHUNK // SPL)]],
                    bufs[bj].at[pl.ds(t * (CHUNK // SPL), CHUNK // SPL)],
                    sems[bj])

        def _gwait(i, b):
            for t in range(SPL):
                pltpu.make_async_copy(
                    v_hbm.at[col_v.at[i, pl.ds(t * (CHUNK // SPL), CHUNK // SPL)]],
                    bufs[b].at[pl.ds(t * (CHUNK // SPL), CHUNK // SPL)],
                    sems[b]).wait()

        for b in range(NBUF - 1):
            _gstart(b, b)

        def _quad(q, _q):
            for b in range(NBUF):
                i = q * NBUF + b
                j = i + NBUF - 1
                bj = (b + NBUF - 1) % NBUF

                @pl.when(j < IB)
                def _():
                    _gstart(j, bj)

                _gwait(i, b)

                def _scale(g, _c):
                    vv = val_v[i, pl.ds(g * 16, 16)]
                    for l in range(16):
                        r = g * 16 + l
                        s = jnp.take(vv, jnp.full((16,), l, jnp.int32))
                        for d in range(D_FEAT // 16):
                            ds = pl.ds(d * 16, 16)
                            bufs[b][r, ds] = bufs[b][r, ds] * s
                    return _c

                lax.fori_loop(0, CHUNK // 16, _scale, None)
                pltpu.sync_copy(bufs[b], acc_sh.at[row_v.at[i]], add=True)
            return _q

        lax.fori_loop(0, IB // NBUF, _quad, None)
        return _

    lax.fori_loop(0, NBLK, _blk, None)

    plsc.subcore_barrier()

    def _dump(i, _):
        j = sid + NS * i

        @pl.when(j < NRCH)
        def _():
            off = pl.multiple_of(j * RCH, 8)
            dbuf = ring.at[pl.ds(0, RCH)]
            pltpu.sync_copy(acc_sh.at[pl.ds(off, RCH)], dbuf)
            hoff = pl.multiple_of(cid * N_NODES + off, 8)
            pltpu.sync_copy(dbuf, p_hbm.at[pl.ds(hoff, RCH)])
        return _

    lax.fori_loop(0, (NRCH + NS - 1) // NS, _dump, None)


def _combine_body(p_hbm, t1_hbm, t0_hbm, s_hbm, par_hbm, tn_hbm, sn_hbm,
                  par_v, bp0, bp1, b1, b0, bs, sem):
    cid = lax.axis_index("c")
    sid = lax.axis_index("s")
    w = cid * NS + sid
    pltpu.sync_copy(par_hbm, par_v)
    pv = par_v[...]

    def _bc(k):
        return jnp.take(pv, jnp.full((16,), k, jnp.int32))

    av, bv, cv, sv, ckv = _bc(0), _bc(1), _bc(2), _bc(3), _bc(4)

    def _it(i, _):
        j = w + NW * i

        @pl.when(j < NRCH)
        def _():
            roff = pl.multiple_of(j * RCH, 8)
            rsl = pl.ds(roff, RCH)
            d0 = pltpu.async_copy(p_hbm.at[rsl], bp0, sem.at[0])
            d1 = pltpu.async_copy(
                p_hbm.at[pl.ds(pl.multiple_of(N_NODES + roff, 8), RCH)],
                bp1, sem.at[1])
            d2 = pltpu.async_copy(t1_hbm.at[rsl], b1, sem.at[2])
            d3 = pltpu.async_copy(t0_hbm.at[rsl], b0, sem.at[3])
            d4 = pltpu.async_copy(s_hbm.at[rsl], bs, sem.at[4])
            d0.wait(); d1.wait(); d2.wait(); d3.wait(); d4.wait()

            def _row(r, _c):
                for v in range(D_FEAT // 16):
                    ds = pl.ds(v * 16, 16)
                    t = (av * (bp0[r, ds] + bp1[r, ds])
                         + bv * b1[r, ds] + cv * b0[r, ds])
                    bp0[r, ds] = t
                    bs[r, ds] = sv * bs[r, ds] + ckv * t
                return _c

            lax.fori_loop(0, RCH, _row, None)
            d5 = pltpu.async_copy(bp0, tn_hbm.at[rsl], sem.at[0])
            d6 = pltpu.async_copy(bs, sn_hbm.at[rsl], sem.at[1])
            d5.wait(); d6.wait()
        return _

    lax.fori_loop(0, ITER, _it, None)


def _make_spmv():
    mesh = plsc.VectorSubcoreMesh(core_axis_name="c", subcore_axis_name="s")
    return pl.kernel(
        _spmv_body,
        out_type=jax.ShapeDtypeStruct((NC * N_NODES, D_FEAT), jnp.float32),
        mesh=mesh,
        scratch_types=[
            pltpu.VMEM((IB, CHUNK), jnp.int32),
            pltpu.VMEM((IB, CHUNK), jnp.int32),
            pltpu.VMEM((IB, CHUNK), jnp.float32),
            pltpu.VMEM((NBUF * CHUNK, D_FEAT), jnp.float32),
            pltpu.SemaphoreType.DMA((NBUF,)),
            pltpu.VMEM_SHARED((ACC_R, D_FEAT), jnp.float32),
        ],
    )


def _make_combine():
    mesh = plsc.VectorSubcoreMesh(core_axis_name="c", subcore_axis_name="s")
    return pl.kernel(
        _combine_body,
        out_type=(jax.ShapeDtypeStruct((N_NODES, D_FEAT), jnp.float32),
                  jax.ShapeDtypeStruct((N_NODES, D_FEAT), jnp.float32)),
        mesh=mesh,
        scratch_types=[
            pltpu.VMEM((16,), jnp.float32),
            pltpu.VMEM((RCH, D_FEAT), jnp.float32),
            pltpu.VMEM((RCH, D_FEAT), jnp.float32),
            pltpu.VMEM((RCH, D_FEAT), jnp.float32),
            pltpu.VMEM((RCH, D_FEAT), jnp.float32),
            pltpu.VMEM((RCH, D_FEAT), jnp.float32),
            pltpu.SemaphoreType.DMA((5,)),
        ],
    )


def kernel(edge_index, edge_values, X):
    coeffs = _cheb(CHEB_ORDER, T_SCALE)
    row = edge_index[0].astype(jnp.int32)
    col = edge_index[1].astype(jnp.int32)
    val = edge_values.astype(jnp.float32)

    pad = EPAD - N_EDGES
    pidx = jnp.arange(pad, dtype=jnp.int32)
    col2 = jnp.concatenate([col, pidx % 8]).reshape(NCH, CHUNK)
    row2 = jnp.concatenate([row, N_NODES + (pidx % 64)]).reshape(NCH, CHUNK)
    val2 = jnp.pad(val, (0, pad)).reshape(NCH, CHUNK)

    spmv = _make_spmv()
    combine = _make_combine()

    def params(a, b, c, s, ck):
        return jnp.asarray([a, b, c, s, ck] + [0.0] * 11, jnp.float32)

    p = spmv(col2, row2, val2, X)
    T1, S = combine(p, X, X, X,
                    params(1.0, -1.0, 0.0, float(coeffs[0]), float(coeffs[1])))
    T0 = X
    for k in range(2, CHEB_ORDER):
        p = spmv(col2, row2, val2, T1)
        Tn, S = combine(p, T1, T0, S,
                        params(2.0, -2.0, -1.0, 1.0, float(coeffs[k])))
        T0, T1 = T1, Tn
    return S


# speedup vs baseline: 1.0306x; 1.0306x over previous
import jax
import jax.numpy as jnp
import numpy as np
from jax import lax
from jax.experimental import pallas as pl
from jax.experimental.pallas import tpu as pltpu
from jax.experimental.pallas import tpu_sc as plsc

CHEB_ORDER = 30
T_SCALE = 5.0
N_NODES = 10000
N_EDGES = 320000
D_FEAT = 128
LAMBDA_MAX = 2.0

NC = 2
NS = 16
NW = NC * NS
CHUNK = 80
NCH = 4096
CPT = NCH // NW
EPAD = NCH * CHUNK
IB = 16
NBLK = CPT // IB
NBUF = 4
ACC_R = 10112
RCH = 40
NRCH = N_NODES // RCH
ITER = (NRCH + NW - 1) // NW


def _cheb(M, t_scale, lambda_max=LAMBDA_MAX):
    j = np.arange(M, dtype=np.float64)
    x = np.cos(np.pi * (j + 0.5) / M)
    lambdas = lambda_max / 2.0 * (x + 1.0)
    f_vals = np.exp(-t_scale * lambdas)
    coeffs = np.zeros(M, dtype=np.float64)
    for k in range(M):
        T_k_x = np.cos(k * np.arccos(x))
        coeffs[k] = 2.0 / M * np.sum(f_vals * T_k_x)
    coeffs[0] /= 2.0
    return coeffs.astype(np.float32)


def _spmv_body(col_hbm, row_hbm, val_hbm, v_hbm, p_hbm,
               col_v, row_v, val_v, ring, sem, acc_sh):
    cid = lax.axis_index("c")
    sid = lax.axis_index("s")
    tile = cid * NS + sid

    zbuf = ring.at[pl.ds(0, RCH)]

    def _zr(r, _):
        for d in range(D_FEAT // 16):
            zbuf[r, pl.ds(d * 16, 16)] = jnp.zeros((16,), jnp.float32)
        return _

    lax.fori_loop(0, RCH, _zr, None)

    def _zc(i, _):
        j = sid + NS * i

        @pl.when(j * RCH < ACC_R)
        def _():
            off = pl.multiple_of(j * RCH, 8)
            pltpu.sync_copy(zbuf, acc_sh.at[pl.ds(off, RCH)])
        return _

    lax.fori_loop(0, (ACC_R // RCH + NS - 1) // NS, _zc, None)

    plsc.subcore_barrier()

    bufs = [ring.at[pl.ds(b * CHUNK, CHUNK)] for b in range(NBUF)]
    sems = [sem.at[b] for b in range(NBUF)]
    tbase = pl.multiple_of(tile * CPT, 8)

    def _blk(bi, _):
        boff = pl.multiple_of(tbase + bi * IB, 8)
        pltpu.sync_copy(col_hbm.at[pl.ds(boff, IB)], col_v)
        pltpu.sync_copy(row_hbm.at[pl.ds(boff, IB)], row_v)
        pltpu.sync_copy(val_hbm.at[pl.ds(boff, IB)], val_v)

        for b in range(NBUF - 1):
            pltpu.async_copy(v_hbm.at[col_v.at[b]], bufs[b], sems[b])

        def _quad(q, _q):
            for b in range(NBUF):
                i = q * NBUF + b
                j = i + NBUF - 1
                bj = (b + NBUF - 1) % NBUF

                @pl.when(j < IB)
                def _():
                    pltpu.async_copy(v_hbm.at[col_v.at[j]], bufs[bj], sems[bj])

                pltpu.make_async_copy(v_hbm.at[col_v.at[i]], bufs[b],
                                      sems[b]).wait()

                def _scale(g, _c):
                    vv = val_v[i, pl.ds(g * 16, 16)]
                    for l in range(16):
                        r = g * 16 + l
                        s = jnp.take(vv, jnp.full((16,), l, jnp.int32))
                        for d in range(D_FEAT // 16):
                            ds = pl.ds(d * 16, 16)
                            bufs[b][r, ds] = bufs[b][r, ds] * s
                    return _c

                lax.fori_loop(0, CHUNK // 16, _scale, None)
                pltpu.sync_copy(bufs[b], acc_sh.at[row_v.at[i]], add=True)
            return _q

        lax.fori_loop(0, IB // NBUF, _quad, None)
        return _

    lax.fori_loop(0, NBLK, _blk, None)

    plsc.subcore_barrier()

    def _dump(i, _):
        j = sid + NS * i

        @pl.when(j < NRCH)
        def _():
            off = pl.multiple_of(j * RCH, 8)
            dbuf = ring.at[pl.ds(0, RCH)]
            pltpu.sync_copy(acc_sh.at[pl.ds(off, RCH)], dbuf)
            hoff = pl.multiple_of(cid * N_NODES + off, 8)
            pltpu.sync_copy(dbuf, p_hbm.at[pl.ds(hoff, RCH)])
        return _

    lax.fori_loop(0, (NRCH + NS - 1) // NS, _dump, None)


def _combine_body(p_hbm, t1_hbm, t0_hbm, s_hbm, par_hbm, tn_hbm, sn_hbm,
                  par_v, bp0, bp1, b1, b0, bs, sem):
    cid = lax.axis_index("c")
    sid = lax.axis_index("s")
    w = cid * NS + sid
    pltpu.sync_copy(par_hbm, par_v)
    pv = par_v[...]

    def _bc(k):
        return jnp.take(pv, jnp.full((16,), k, jnp.int32))

    av, bv, cv, sv, ckv = _bc(0), _bc(1), _bc(2), _bc(3), _bc(4)

    def _it(i, _):
        j = w + NW * i

        @pl.when(j < NRCH)
        def _():
            roff = pl.multiple_of(j * RCH, 8)
            rsl = pl.ds(roff, RCH)
            d0 = pltpu.async_copy(p_hbm.at[rsl], bp0, sem.at[0])
            d1 = pltpu.async_copy(
                p_hbm.at[pl.ds(pl.multiple_of(N_NODES + roff, 8), RCH)],
                bp1, sem.at[1])
            d2 = pltpu.async_copy(t1_hbm.at[rsl], b1, sem.at[2])
            d3 = pltpu.async_copy(t0_hbm.at[rsl], b0, sem.at[3])
            d4 = pltpu.async_copy(s_hbm.at[rsl], bs, sem.at[4])
            d0.wait(); d1.wait(); d2.wait(); d3.wait(); d4.wait()

            def _row(r, _c):
                for v in range(D_FEAT // 16):
                    ds = pl.ds(v * 16, 16)
                    t = (av * (bp0[r, ds] + bp1[r, ds])
                         + bv * b1[r, ds] + cv * b0[r, ds])
                    bp0[r, ds] = t
                    bs[r, ds] = sv * bs[r, ds] + ckv * t
                return _c

            lax.fori_loop(0, RCH, _row, None)
            d5 = pltpu.async_copy(bp0, tn_hbm.at[rsl], sem.at[0])
            d6 = pltpu.async_copy(bs, sn_hbm.at[rsl], sem.at[1])
            d5.wait(); d6.wait()
        return _

    lax.fori_loop(0, ITER, _it, None)


def _make_spmv():
    mesh = plsc.VectorSubcoreMesh(core_axis_name="c", subcore_axis_name="s")
    return pl.kernel(
        _spmv_body,
        out_type=jax.ShapeDtypeStruct((NC * N_NODES, D_FEAT), jnp.float32),
        mesh=mesh,
        scratch_types=[
            pltpu.VMEM((IB, CHUNK), jnp.int32),
            pltpu.VMEM((IB, CHUNK), jnp.int32),
            pltpu.VMEM((IB, CHUNK), jnp.float32),
            pltpu.VMEM((NBUF * CHUNK, D_FEAT), jnp.float32),
            pltpu.SemaphoreType.DMA((NBUF,)),
            pltpu.VMEM_SHARED((ACC_R, D_FEAT), jnp.float32),
        ],
    )


def _make_combine():
    mesh = plsc.VectorSubcoreMesh(core_axis_name="c", subcore_axis_name="s")
    return pl.kernel(
        _combine_body,
        out_type=(jax.ShapeDtypeStruct((N_NODES, D_FEAT), jnp.float32),
                  jax.ShapeDtypeStruct((N_NODES, D_FEAT), jnp.float32)),
        mesh=mesh,
        scratch_types=[
            pltpu.VMEM((16,), jnp.float32),
            pltpu.VMEM((RCH, D_FEAT), jnp.float32),
            pltpu.VMEM((RCH, D_FEAT), jnp.float32),
            pltpu.VMEM((RCH, D_FEAT), jnp.float32),
            pltpu.VMEM((RCH, D_FEAT), jnp.float32),
            pltpu.VMEM((RCH, D_FEAT), jnp.float32),
            pltpu.SemaphoreType.DMA((5,)),
        ],
    )


def kernel(edge_index, edge_values, X):
    coeffs = _cheb(CHEB_ORDER, T_SCALE)
    row = edge_index[0].astype(jnp.int32)
    col = edge_index[1].astype(jnp.int32)
    val = edge_values.astype(jnp.float32)

    pad = EPAD - N_EDGES
    pidx = jnp.arange(pad, dtype=jnp.int32)
    col2 = jnp.concatenate([col, pidx % 8]).reshape(NCH, CHUNK)
    row2 = jnp.concatenate([row, N_NODES + (pidx % 64)]).reshape(NCH, CHUNK)
    val2 = jnp.pad(val, (0, pad)).reshape(NCH, CHUNK)

    spmv = _make_spmv()
    combine = _make_combine()

    def params(a, b, c, s, ck):
        return jnp.asarray([a, b, c, s, ck] + [0.0] * 11, jnp.float32)

    p = spmv(col2, row2, val2, X)
    T1, S = combine(p, X, X, X,
                    params(1.0, -1.0, 0.0, float(coeffs[0]), float(coeffs[1])))
    T0 = X
    for k in range(2, CHEB_ORDER):
        p = spmv(col2, row2, val2, T1)
        Tn, S = combine(p, T1, T0, S,
                        params(2.0, -2.0, -1.0, 1.0, float(coeffs[k])))
        T0, T1 = T1, Tn
    return S


# speedup vs baseline: 1.1082x; 1.0753x over previous
import jax
import jax.numpy as jnp
import numpy as np
from jax import lax
from jax.experimental import pallas as pl
from jax.experimental.pallas import tpu as pltpu
from jax.experimental.pallas import tpu_sc as plsc

CHEB_ORDER = 30
T_SCALE = 5.0
N_NODES = 10000
N_EDGES = 320000
D_FEAT = 128
LAMBDA_MAX = 2.0

NC = 2
NS = 16
NW = NC * NS
CHUNK = 64
NCH = 5120
CPT = NCH // NW
EPAD = NCH * CHUNK
IB = 32
NBLK = CPT // IB
NBUF = 4
ACC_R = 10112
RCH = 40
NRCH = N_NODES // RCH
ITER = (NRCH + NW - 1) // NW


def _cheb(M, t_scale, lambda_max=LAMBDA_MAX):
    j = np.arange(M, dtype=np.float64)
    x = np.cos(np.pi * (j + 0.5) / M)
    lambdas = lambda_max / 2.0 * (x + 1.0)
    f_vals = np.exp(-t_scale * lambdas)
    coeffs = np.zeros(M, dtype=np.float64)
    for k in range(M):
        T_k_x = np.cos(k * np.arccos(x))
        coeffs[k] = 2.0 / M * np.sum(f_vals * T_k_x)
    coeffs[0] /= 2.0
    return coeffs.astype(np.float32)


def _spmv_body(col_hbm, row_hbm, val_hbm, v_hbm, p_hbm,
               col_v, row_v, val_v, ring, sem, acc_sh):
    cid = lax.axis_index("c")
    sid = lax.axis_index("s")
    tile = cid * NS + sid

    zbuf = ring.at[pl.ds(0, RCH)]

    def _zr(r, _):
        for d in range(D_FEAT // 16):
            zbuf[r, pl.ds(d * 16, 16)] = jnp.zeros((16,), jnp.float32)
        return _

    lax.fori_loop(0, RCH, _zr, None)

    def _zc(i, _):
        j = sid + NS * i

        @pl.when(j * RCH < ACC_R)
        def _():
            off = pl.multiple_of(j * RCH, 8)
            pltpu.sync_copy(zbuf, acc_sh.at[pl.ds(off, RCH)])
        return _

    lax.fori_loop(0, (ACC_R // RCH + NS - 1) // NS, _zc, None)

    plsc.subcore_barrier()

    bufs = [ring.at[pl.ds(b * CHUNK, CHUNK)] for b in range(NBUF)]
    sems = [sem.at[b] for b in range(NBUF)]
    tbase = pl.multiple_of(tile * CPT, 8)

    def _blk(bi, _):
        boff = pl.multiple_of(tbase + bi * IB, 8)
        pltpu.sync_copy(col_hbm.at[pl.ds(boff, IB)], col_v)
        pltpu.sync_copy(row_hbm.at[pl.ds(boff, IB)], row_v)
        pltpu.sync_copy(val_hbm.at[pl.ds(boff, IB)], val_v)

        for b in range(NBUF - 1):
            pltpu.async_copy(v_hbm.at[col_v.at[b]], bufs[b], sems[b])

        def _quad(q, _q):
            for b in range(NBUF):
                i = q * NBUF + b
                j = i + NBUF - 1
                bj = (b + NBUF - 1) % NBUF

                @pl.when(j < IB)
                def _():
                    pltpu.async_copy(v_hbm.at[col_v.at[j]], bufs[bj], sems[bj])

                pltpu.make_async_copy(v_hbm.at[col_v.at[i]], bufs[b],
                                      sems[b]).wait()

                def _scale(g, _c):
                    vv = val_v[i, pl.ds(g * 16, 16)]
                    for l in range(16):
                        r = g * 16 + l
                        s = jnp.take(vv, jnp.full((16,), l, jnp.int32))
                        for d in range(D_FEAT // 16):
                            ds = pl.ds(d * 16, 16)
                            bufs[b][r, ds] = bufs[b][r, ds] * s
                    return _c

                lax.fori_loop(0, CHUNK // 16, _scale, None)
                pltpu.sync_copy(bufs[b], acc_sh.at[row_v.at[i]], add=True)
            return _q

        lax.fori_loop(0, IB // NBUF, _quad, None)
        return _

    lax.fori_loop(0, NBLK, _blk, None)

    plsc.subcore_barrier()

    def _dump(i, _):
        j = sid + NS * i

        @pl.when(j < NRCH)
        def _():
            off = pl.multiple_of(j * RCH, 8)
            dbuf = ring.at[pl.ds(0, RCH)]
            pltpu.sync_copy(acc_sh.at[pl.ds(off, RCH)], dbuf)
            hoff = pl.multiple_of(cid * N_NODES + off, 8)
            pltpu.sync_copy(dbuf, p_hbm.at[pl.ds(hoff, RCH)])
        return _

    lax.fori_loop(0, (NRCH + NS - 1) // NS, _dump, None)


def _combine_body(p_hbm, t1_hbm, t0_hbm, s_hbm, par_hbm, tn_hbm, sn_hbm,
                  par_v, bp0, bp1, b1, b0, bs, sem):
    cid = lax.axis_index("c")
    sid = lax.axis_index("s")
    w = cid * NS + sid
    pltpu.sync_copy(par_hbm, par_v)
    pv = par_v[...]

    def _bc(k):
        return jnp.take(pv, jnp.full((16,), k, jnp.int32))

    av, bv, cv, sv, ckv = _bc(0), _bc(1), _bc(2), _bc(3), _bc(4)

    def _it(i, _):
        j = w + NW * i

        @pl.when(j < NRCH)
        def _():
            roff = pl.multiple_of(j * RCH, 8)
            rsl = pl.ds(roff, RCH)
            d0 = pltpu.async_copy(p_hbm.at[rsl], bp0, sem.at[0])
            d1 = pltpu.async_copy(
                p_hbm.at[pl.ds(pl.multiple_of(N_NODES + roff, 8), RCH)],
                bp1, sem.at[1])
            d2 = pltpu.async_copy(t1_hbm.at[rsl], b1, sem.at[2])
            d3 = pltpu.async_copy(t0_hbm.at[rsl], b0, sem.at[3])
            d4 = pltpu.async_copy(s_hbm.at[rsl], bs, sem.at[4])
            d0.wait(); d1.wait(); d2.wait(); d3.wait(); d4.wait()

            def _row(r, _c):
                for v in range(D_FEAT // 16):
                    ds = pl.ds(v * 16, 16)
                    t = (av * (bp0[r, ds] + bp1[r, ds])
                         + bv * b1[r, ds] + cv * b0[r, ds])
                    bp0[r, ds] = t
                    bs[r, ds] = sv * bs[r, ds] + ckv * t
                return _c

            lax.fori_loop(0, RCH, _row, None)
            d5 = pltpu.async_copy(bp0, tn_hbm.at[rsl], sem.at[0])
            d6 = pltpu.async_copy(bs, sn_hbm.at[rsl], sem.at[1])
            d5.wait(); d6.wait()
        return _

    lax.fori_loop(0, ITER, _it, None)


def _make_spmv():
    mesh = plsc.VectorSubcoreMesh(core_axis_name="c", subcore_axis_name="s")
    return pl.kernel(
        _spmv_body,
        out_type=jax.ShapeDtypeStruct((NC * N_NODES, D_FEAT), jnp.float32),
        mesh=mesh,
        scratch_types=[
            pltpu.VMEM((IB, CHUNK), jnp.int32),
            pltpu.VMEM((IB, CHUNK), jnp.int32),
            pltpu.VMEM((IB, CHUNK), jnp.float32),
            pltpu.VMEM((NBUF * CHUNK, D_FEAT), jnp.float32),
            pltpu.SemaphoreType.DMA((NBUF,)),
            pltpu.VMEM_SHARED((ACC_R, D_FEAT), jnp.float32),
        ],
    )


def _make_combine():
    mesh = plsc.VectorSubcoreMesh(core_axis_name="c", subcore_axis_name="s")
    return pl.kernel(
        _combine_body,
        out_type=(jax.ShapeDtypeStruct((N_NODES, D_FEAT), jnp.float32),
                  jax.ShapeDtypeStruct((N_NODES, D_FEAT), jnp.float32)),
        mesh=mesh,
        scratch_types=[
            pltpu.VMEM((16,), jnp.float32),
            pltpu.VMEM((RCH, D_FEAT), jnp.float32),
            pltpu.VMEM((RCH, D_FEAT), jnp.float32),
            pltpu.VMEM((RCH, D_FEAT), jnp.float32),
            pltpu.VMEM((RCH, D_FEAT), jnp.float32),
            pltpu.VMEM((RCH, D_FEAT), jnp.float32),
            pltpu.SemaphoreType.DMA((5,)),
        ],
    )


def kernel(edge_index, edge_values, X):
    coeffs = _cheb(CHEB_ORDER, T_SCALE)
    row = edge_index[0].astype(jnp.int32)
    col = edge_index[1].astype(jnp.int32)
    val = edge_values.astype(jnp.float32)

    pad = EPAD - N_EDGES
    pidx = jnp.arange(pad, dtype=jnp.int32)
    col2 = jnp.concatenate([col, pidx % 8]).reshape(NCH, CHUNK)
    row2 = jnp.concatenate([row, N_NODES + (pidx % 64)]).reshape(NCH, CHUNK)
    val2 = jnp.pad(val, (0, pad)).reshape(NCH, CHUNK)

    spmv = _make_spmv()
    combine = _make_combine()

    def params(a, b, c, s, ck):
        return jnp.asarray([a, b, c, s, ck] + [0.0] * 11, jnp.float32)

    p = spmv(col2, row2, val2, X)
    T1, S = combine(p, X, X, X,
                    params(1.0, -1.0, 0.0, float(coeffs[0]), float(coeffs[1])))
    T0 = X
    for k in range(2, CHEB_ORDER):
        p = spmv(col2, row2, val2, T1)
        Tn, S = combine(p, T1, T0, S,
                        params(2.0, -2.0, -1.0, 1.0, float(coeffs[k])))
        T0, T1 = T1, Tn
    return S


# speedup vs baseline: 1.1201x; 1.0107x over previous
import jax
import jax.numpy as jnp
import numpy as np
from jax import lax
from jax.experimental import pallas as pl
from jax.experimental.pallas import tpu as pltpu
from jax.experimental.pallas import tpu_sc as plsc

CHEB_ORDER = 30
T_SCALE = 5.0
N_NODES = 10000
N_EDGES = 320000
D_FEAT = 128
LAMBDA_MAX = 2.0

NC = 2
NS = 16
NW = NC * NS
CHUNK = 64
NCH = 5120
CPT = NCH // NW
EPAD = NCH * CHUNK
IB = 32
NBLK = CPT // IB
NBUF = 4
ACC_R = 10112
RCH = 40
NRCH = N_NODES // RCH
ITER = (NRCH + NW - 1) // NW


def _cheb(M, t_scale, lambda_max=LAMBDA_MAX):
    j = np.arange(M, dtype=np.float64)
    x = np.cos(np.pi * (j + 0.5) / M)
    lambdas = lambda_max / 2.0 * (x + 1.0)
    f_vals = np.exp(-t_scale * lambdas)
    coeffs = np.zeros(M, dtype=np.float64)
    for k in range(M):
        T_k_x = np.cos(k * np.arccos(x))
        coeffs[k] = 2.0 / M * np.sum(f_vals * T_k_x)
    coeffs[0] /= 2.0
    return coeffs.astype(np.float32)


def _spmv_body(col_hbm, row_hbm, val_hbm, v_hbm, p_hbm,
               col_v, row_v, val_v, ring, sem, acc_sh):
    cid = lax.axis_index("c")
    sid = lax.axis_index("s")
    tile = cid * NS + sid

    zbuf = ring.at[pl.ds(0, RCH)]

    def _zr(r, _):
        for d in range(D_FEAT // 16):
            zbuf[r, pl.ds(d * 16, 16)] = jnp.zeros((16,), jnp.float32)
        return _

    lax.fori_loop(0, RCH, _zr, None)

    def _zc(i, _):
        j = sid + NS * i

        @pl.when(j * RCH < ACC_R)
        def _():
            off = pl.multiple_of(j * RCH, 8)
            pltpu.sync_copy(zbuf, acc_sh.at[pl.ds(off, RCH)])
        return _

    lax.fori_loop(0, (ACC_R // RCH + NS - 1) // NS, _zc, None)

    plsc.subcore_barrier()

    bufs = [ring.at[pl.ds(b * CHUNK, CHUNK)] for b in range(NBUF)]
    sems = [sem.at[b] for b in range(NBUF)]
    tbase = pl.multiple_of(tile * CPT, 8)

    def _blk(bi, _):
        boff = pl.multiple_of(tbase + bi * IB, 8)
        pltpu.sync_copy(col_hbm.at[pl.ds(boff, IB)], col_v)
        pltpu.sync_copy(row_hbm.at[pl.ds(boff, IB)], row_v)
        pltpu.sync_copy(val_hbm.at[pl.ds(boff, IB)], val_v)

        for b in range(NBUF - 1):
            pltpu.async_copy(v_hbm.at[col_v.at[b]], bufs[b], sems[b])

        def _quad(q, _q):
            for b in range(NBUF):
                i = q * NBUF + b
                j = i + NBUF - 1
                bj = (b + NBUF - 1) % NBUF

                @pl.when(j < IB)
                def _():
                    pltpu.async_copy(v_hbm.at[col_v.at[j]], bufs[bj], sems[bj])

                pltpu.make_async_copy(v_hbm.at[col_v.at[i]], bufs[b],
                                      sems[b]).wait()

                def _scale(g, _c):
                    vv = val_v[i, pl.ds(g * 16, 16)]
                    for l in range(16):
                        r = g * 16 + l
                        s = jnp.take(vv, jnp.full((16,), l, jnp.int32))
                        for d in range(D_FEAT // 16):
                            ds = pl.ds(d * 16, 16)
                            bufs[b][r, ds] = bufs[b][r, ds] * s
                    return _c

                lax.fori_loop(0, CHUNK // 16, _scale, None)
                pltpu.sync_copy(bufs[b], acc_sh.at[row_v.at[i]], add=True)
            return _q

        lax.fori_loop(0, IB // NBUF, _quad, None)
        return _

    lax.fori_loop(0, NBLK, _blk, None)

    plsc.subcore_barrier()

    def _dump(i, _):
        j = sid + NS * i

        @pl.when(j < NRCH)
        def _():
            off = pl.multiple_of(j * RCH, 8)
            dbuf = ring.at[pl.ds(0, RCH)]
            pltpu.sync_copy(acc_sh.at[pl.ds(off, RCH)], dbuf)
            hoff = pl.multiple_of(cid * N_NODES + off, 8)
            pltpu.sync_copy(dbuf, p_hbm.at[pl.ds(hoff, RCH)])
        return _

    lax.fori_loop(0, (NRCH + NS - 1) // NS, _dump, None)


def _combine_body(p_hbm, t1_hbm, t0_hbm, s_hbm, par_hbm, tn_hbm, sn_hbm,
                  par_v, bp0, bp1, b1, b0, bs, sem):
    cid = lax.axis_index("c")
    sid = lax.axis_index("s")
    w = cid * NS + sid
    pltpu.sync_copy(par_hbm, par_v)
    pv = par_v[...]

    def _bc(k):
        return jnp.take(pv, jnp.full((16,), k, jnp.int32))

    av, bv, cv, sv, ckv = _bc(0), _bc(1), _bc(2), _bc(3), _bc(4)

    def _it(i, _):
        j = w + NW * i

        @pl.when(j < NRCH)
        def _():
            roff = pl.multiple_of(j * RCH, 8)
            rsl = pl.ds(roff, RCH)
            d0 = pltpu.async_copy(p_hbm.at[rsl], bp0, sem.at[0])
            d1 = pltpu.async_copy(
                p_hbm.at[pl.ds(pl.multiple_of(N_NODES + roff, 8), RCH)],
                bp1, sem.at[1])
            d2 = pltpu.async_copy(t1_hbm.at[rsl], b1, sem.at[2])
            d3 = pltpu.async_copy(t0_hbm.at[rsl], b0, sem.at[3])
            d4 = pltpu.async_copy(s_hbm.at[rsl], bs, sem.at[4])
            d0.wait(); d1.wait(); d2.wait(); d3.wait(); d4.wait()

            def _row(r, _c):
                for v in range(D_FEAT // 16):
                    ds = pl.ds(v * 16, 16)
                    t = (av * (bp0[r, ds] + bp1[r, ds])
                         + bv * b1[r, ds] + cv * b0[r, ds])
                    bp0[r, ds] = t
                    bs[r, ds] = sv * bs[r, ds] + ckv * t
                return _c

            lax.fori_loop(0, RCH, _row, None)
            d5 = pltpu.async_copy(bp0, tn_hbm.at[rsl], sem.at[0])
            d6 = pltpu.async_copy(bs, sn_hbm.at[rsl], sem.at[1])
            d5.wait(); d6.wait()
        return _

    lax.fori_loop(0, ITER, _it, None)


def _make_spmv():
    mesh = plsc.VectorSubcoreMesh(core_axis_name="c", subcore_axis_name="s")
    return pl.kernel(
        _spmv_body,
        out_type=jax.ShapeDtypeStruct((NC * N_NODES, D_FEAT), jnp.float32),
        mesh=mesh,
        scratch_types=[
            pltpu.VMEM((IB, CHUNK), jnp.int32),
            pltpu.VMEM((IB, CHUNK), jnp.int32),
            pltpu.VMEM((IB, CHUNK), jnp.float32),
            pltpu.VMEM((NBUF * CHUNK, D_FEAT), jnp.float32),
            pltpu.SemaphoreType.DMA((NBUF,)),
            pltpu.VMEM_SHARED((ACC_R, D_FEAT), jnp.float32),
        ],
    )


def _make_combine():
    mesh = plsc.VectorSubcoreMesh(core_axis_name="c", subcore_axis_name="s")
    return pl.kernel(
        _combine_body,
        out_type=(jax.ShapeDtypeStruct((N_NODES, D_FEAT), jnp.float32),
                  jax.ShapeDtypeStruct((N_NODES, D_FEAT), jnp.float32)),
        mesh=mesh,
        scratch_types=[
            pltpu.VMEM((16,), jnp.float32),
            pltpu.VMEM((RCH, D_FEAT), jnp.float32),
            pltpu.VMEM((RCH, D_FEAT), jnp.float32),
            pltpu.VMEM((RCH, D_FEAT), jnp.float32),
            pltpu.VMEM((RCH, D_FEAT), jnp.float32),
            pltpu.VMEM((RCH, D_FEAT), jnp.float32),
            pltpu.SemaphoreType.DMA((5,)),
        ],
    )


def kernel(edge_index, edge_values, X):
    coeffs = _cheb(CHEB_ORDER, T_SCALE)
    row = edge_index[0].astype(jnp.int32)
    col = edge_index[1].astype(jnp.int32)
    val = edge_values.astype(jnp.float32)

    pad = EPAD - N_EDGES
    pidx = jnp.arange(pad, dtype=jnp.int32)
    col2 = jnp.concatenate([col, (pidx * 131) % N_NODES]).reshape(NCH, CHUNK)
    row2 = jnp.concatenate([row, N_NODES + (pidx % 64)]).reshape(NCH, CHUNK)
    val2 = jnp.pad(val, (0, pad)).reshape(NCH, CHUNK)

    spmv = _make_spmv()
    combine = _make_combine()

    def params(a, b, c, s, ck):
        return jnp.asarray([a, b, c, s, ck] + [0.0] * 11, jnp.float32)

    p = spmv(col2, row2, val2, X)
    T1, S = combine(p, X, X, X,
                    params(1.0, -1.0, 0.0, float(coeffs[0]), float(coeffs[1])))
    T0 = X
    for k in range(2, CHEB_ORDER):
        p = spmv(col2, row2, val2, T1)
        Tn, S = combine(p, T1, T0, S,
                        params(2.0, -2.0, -1.0, 1.0, float(coeffs[k])))
        T0, T1 = T1, Tn
    return S
